# baseline (speedup 1.0000x reference)
import functools

import jax
import jax.numpy as jnp
from jax import lax
from jax.experimental import pallas as pl
from jax.experimental.pallas import tpu as pltpu
from jax.experimental.pallas import tpu_sc as plsc

N = 10000
E = 160000
D_IN = 128
D_H = 512

NC = 2
NS = 16
NPAD = 10112
JUNK = N
ZROWS = NPAD // NS

NBLK = 80
HBLK = NBLK // 2
EPAD = NS * NBLK * 128
NBLK0 = EPAD // (NC * NS * 128)

BN = 1000
GRID = N // BN

_f32 = jnp.float32
_i32 = jnp.int32


def _sc_mesh():
    return plsc.VectorSubcoreMesh(core_axis_name="c", subcore_axis_name="s")


def _deg_kernel(dst_hbm, ones_hbm, zeros_hbm, out_hbm, idx_v, ones_v, acc):
    c = lax.axis_index("c")
    s = lax.axis_index("s")
    wid = c * NS + s
    pltpu.sync_copy(dst_hbm.at[wid], idx_v)
    pltpu.sync_copy(ones_hbm, ones_v)
    pltpu.sync_copy(zeros_hbm, acc.at[pl.ds(s * ZROWS, ZROWS)])
    plsc.subcore_barrier()

    def body(j, carry):
        pltpu.sync_copy(ones_v, acc.at[idx_v.at[j]], add=True)
        return carry

    lax.fori_loop(0, NBLK0, body, 0)
    plsc.subcore_barrier()
    pltpu.sync_copy(acc.at[pl.ds(s * ZROWS, ZROWS)],
                    out_hbm.at[c, pl.ds(s * ZROWS, ZROWS)])


def _deg(dst_pad):
    ones = jnp.ones((128, 128), _f32)
    zeros = jnp.zeros((ZROWS, 128), _f32)
    k = pl.kernel(
        _deg_kernel,
        out_type=jax.ShapeDtypeStruct((NC, NPAD, 128), _f32),
        mesh=_sc_mesh(),
        scratch_types=[
            pltpu.VMEM((NBLK0, 128), _i32),
            pltpu.VMEM((128, 128), _f32),
            pltpu.VMEM_SHARED((NPAD, 128), _f32),
        ],
    )
    return k(dst_pad.reshape(NC * NS, NBLK0, 128), ones, zeros)


def _seg_body(nc, table_hbm, src_hbm, dst_hbm, zeros_hbm, out_hbm,
              srcv, dstv, buf0, buf1, acc, gsem0, gsem1, ssem0, ssem1):
    c = lax.axis_index("c")
    s = lax.axis_index("s")
    pltpu.sync_copy(dst_hbm.at[s], dstv)
    for r in range(nc // NC):
        ch = r * NC + c
        pltpu.sync_copy(zeros_hbm, acc.at[pl.ds(s * ZROWS, ZROWS)])
        plsc.subcore_barrier()
        for half in range(2):
            base = half * HBLK
            pltpu.sync_copy(src_hbm.at[ch, s, half], srcv)
            def gstart(j, buf, semA, semB):
                pltpu.make_async_copy(
                    table_hbm.at[srcv.at[j, pl.ds(0, 64)]],
                    buf.at[pl.ds(0, 64)], semA).start()
                pltpu.make_async_copy(
                    table_hbm.at[srcv.at[j, pl.ds(64, 64)]],
                    buf.at[pl.ds(64, 64)], semB).start()

            def gwait(j, buf, semA, semB):
                pltpu.make_async_copy(
                    table_hbm.at[srcv.at[j, pl.ds(0, 64)]],
                    buf.at[pl.ds(0, 64)], semA).wait()
                pltpu.make_async_copy(
                    table_hbm.at[srcv.at[j, pl.ds(64, 64)]],
                    buf.at[pl.ds(64, 64)], semB).wait()

            gstart(0, buf0, gsem0, ssem0)
            gstart(1, buf1, gsem1, ssem1)

            def body(t, carry):
                j = 2 * t
                gwait(j, buf0, gsem0, ssem0)
                pltpu.sync_copy(buf0, acc.at[dstv.at[base + j]], add=True)

                @pl.when(j + 2 < HBLK)
                def _():
                    gstart(j + 2, buf0, gsem0, ssem0)

                gwait(j + 1, buf1, gsem1, ssem1)
                pltpu.sync_copy(buf1, acc.at[dstv.at[base + j + 1]], add=True)

                @pl.when(j + 3 < HBLK)
                def _():
                    gstart(j + 3, buf1, gsem1, ssem1)

                return carry

            lax.fori_loop(0, HBLK // 2, body, 0)
        plsc.subcore_barrier()
        pltpu.sync_copy(acc.at[pl.ds(s * ZROWS, ZROWS)],
                        out_hbm.at[ch, pl.ds(s * ZROWS, ZROWS)])
        plsc.subcore_barrier()


def _seg(table, src_pad, dst_pad, nc):
    zeros = jnp.zeros((ZROWS, 128), _f32)
    k = pl.kernel(
        functools.partial(_seg_body, nc),
        out_type=jax.ShapeDtypeStruct((nc, NPAD, 128), _f32),
        mesh=_sc_mesh(),
        scratch_types=[
            pltpu.VMEM((HBLK, 128), _i32),
            pltpu.VMEM((NBLK, 128), _i32),
            pltpu.VMEM((128, 128), _f32),
            pltpu.VMEM((128, 128), _f32),
            pltpu.VMEM_SHARED((NPAD, 128), _f32),
            pltpu.SemaphoreType.DMA,
            pltpu.SemaphoreType.DMA,
            pltpu.SemaphoreType.DMA,
            pltpu.SemaphoreType.DMA,
        ],
    )
    return k(table, src_pad.reshape(nc, NS, 2, HBLK, 128),
             dst_pad.reshape(NS, NBLK, 128), zeros)


def _tc1_body(x_ref, hist_ref, wa_ref, ba_ref, t1_ref, dinv_ref):
    hist = hist_ref[...]
    deg = (jnp.sum(hist[0], axis=-1) + jnp.sum(hist[1], axis=-1)) * 0.0078125 + 1.0
    dinv = lax.rsqrt(deg)
    x = x_ref[...]
    a = jnp.dot(x, wa_ref[...], preferred_element_type=_f32,
                precision=lax.Precision.DEFAULT) + ba_ref[...][None, :]
    m = jnp.max(a, axis=-1, keepdims=True)
    e = jnp.exp(a - m)
    att = e / jnp.sum(e, axis=-1, keepdims=True)
    t1_ref[:, 0, :] = x * dinv[:, None]
    t1_ref[:, 1, :] = (x * att) * dinv[:, None]
    dinv_ref[...] = dinv[:, None]


def _tc1(x, hist, w_att, b_att):
    return pl.pallas_call(
        _tc1_body,
        grid=(GRID,),
        in_specs=[
            pl.BlockSpec((BN, D_IN), lambda i: (i, 0)),
            pl.BlockSpec((NC, BN, 128), lambda i: (0, i, 0)),
            pl.BlockSpec((D_IN, D_IN), lambda i: (0, 0)),
            pl.BlockSpec((D_IN,), lambda i: (0,)),
        ],
        out_specs=[
            pl.BlockSpec((BN, 2, D_IN), lambda i: (i, 0, 0)),
            pl.BlockSpec((BN, 1), lambda i: (i, 0)),
        ],
        out_shape=[
            jax.ShapeDtypeStruct((N, 2, D_IN), _f32),
            jax.ShapeDtypeStruct((N, 1), _f32),
        ],
    )(x, hist, w_att, b_att)


def _tc2_body(agg_ref, t1_ref, dinv_ref, w1_ref, b1_ref, t2_ref, *, enc):
    dinv = dinv_ref[...]
    g = (agg_ref[0] + t1_ref[:, enc, :]) * dinv
    h = jnp.dot(g, w1_ref[...], preferred_element_type=_f32,
                precision=lax.Precision.DEFAULT) + b1_ref[...][None, :]
    h = jnp.maximum(h, 0.0) * dinv
    for cch in range(4):
        t2_ref[:, cch, :] = h[:, cch * 128:(cch + 1) * 128]


def _tc2(agg1, t1, dinv, w1, b1, enc):
    return pl.pallas_call(
        functools.partial(_tc2_body, enc=enc),
        grid=(GRID,),
        in_specs=[
            pl.BlockSpec((1, BN, D_IN), lambda i, e=enc: (e, i, 0)),
            pl.BlockSpec((BN, 2, D_IN), lambda i: (i, 0, 0)),
            pl.BlockSpec((BN, 1), lambda i: (i, 0)),
            pl.BlockSpec((D_IN, D_H), lambda i: (0, 0)),
            pl.BlockSpec((D_H,), lambda i: (0,)),
        ],
        out_specs=pl.BlockSpec((BN, 4, 128), lambda i: (i, 0, 0)),
        out_shape=jax.ShapeDtypeStruct((N, 4, 128), _f32),
    )(agg1, t1, dinv, w1, b1)


def _tc3_body(agg_ref, t2_ref, dinv_ref, w2_ref, b2_ref, wp1_ref, bp1_ref,
              wp2_ref, bp2_ref, z_ref):
    dinv = dinv_ref[...]
    cols = [agg_ref[cch] + t2_ref[:, cch, :] for cch in range(4)]
    g = jnp.concatenate(cols, axis=-1) * dinv
    h2 = jnp.maximum(
        jnp.dot(g, w2_ref[...], preferred_element_type=_f32,
                precision=lax.Precision.DEFAULT) + b2_ref[...][None, :], 0.0)
    p = jnp.maximum(
        jnp.dot(h2, wp1_ref[...], preferred_element_type=_f32,
                precision=lax.Precision.DEFAULT) + bp1_ref[...][None, :], 0.0)
    z = jnp.dot(p, wp2_ref[...], preferred_element_type=_f32,
                precision=lax.Precision.DEFAULT) + bp2_ref[...][None, :]
    z_ref[...] = z


def _tc3(agg2, t2, dinv, w2, b2, wp1, bp1, wp2, bp2):
    return pl.pallas_call(
        _tc3_body,
        grid=(GRID,),
        in_specs=[
            pl.BlockSpec((4, BN, 128), lambda i: (0, i, 0)),
            pl.BlockSpec((BN, 4, 128), lambda i: (i, 0, 0)),
            pl.BlockSpec((BN, 1), lambda i: (i, 0)),
            pl.BlockSpec((D_H, D_H), lambda i: (0, 0)),
            pl.BlockSpec((D_H,), lambda i: (0,)),
            pl.BlockSpec((D_H, D_H), lambda i: (0, 0)),
            pl.BlockSpec((D_H,), lambda i: (0,)),
            pl.BlockSpec((D_H, D_H), lambda i: (0, 0)),
            pl.BlockSpec((D_H,), lambda i: (0,)),
        ],
        out_specs=pl.BlockSpec((BN, D_H), lambda i: (i, 0)),
        out_shape=jax.ShapeDtypeStruct((N, D_H), _f32),
    )(agg2, t2, dinv, w2, b2, wp1, bp1, wp2, bp2)


def kernel(x, edge_index, W_att, b_att, W1, b1, W2, b2, Wp1, bp1, Wp2, bp2):
    src = edge_index[0]
    dst = edge_index[1]
    pad = EPAD - E
    dst_pad = jnp.concatenate([dst, jnp.full((pad,), JUNK, _i32)])
    src_pad = jnp.concatenate([src, jnp.zeros((pad,), _i32)])
    idx1 = src_pad[None, :] * 2 + jnp.arange(2, dtype=_i32)[:, None]
    idx2 = src_pad[None, :] * 4 + jnp.arange(4, dtype=_i32)[:, None]

    hist = _deg(dst_pad)
    t1, dinv = _tc1(x, hist, W_att, b_att)
    agg1 = _seg(t1.reshape(N * 2, 128), idx1, dst_pad, 2)
    z = []
    for enc in range(2):
        t2e = _tc2(agg1, t1, dinv, W1, b1, enc)
        agg2e = _seg(t2e.reshape(N * 4, 128), idx2, dst_pad, 4)
        z.append(_tc3(agg2e, t2e, dinv, W2, b2, Wp1, bp1, Wp2, bp2))
    return (z[0], z[1])


# speedup vs baseline: 1.0511x; 1.0511x over previous
import functools

import jax
import jax.numpy as jnp
from jax import lax
from jax.experimental import pallas as pl
from jax.experimental.pallas import tpu as pltpu
from jax.experimental.pallas import tpu_sc as plsc

N = 10000
E = 160000
D_IN = 128
D_H = 512

NC = 2
NS = 16
NPAD = 10112
JUNK = N
ZROWS = NPAD // NS

NBLK = 80
HBLK = NBLK // 2
EPAD = NS * NBLK * 128
NBLK0 = EPAD // (NC * NS * 128)

BN = 1000
GRID = N // BN

_f32 = jnp.float32
_i32 = jnp.int32


def _sc_mesh():
    return plsc.VectorSubcoreMesh(core_axis_name="c", subcore_axis_name="s")


def _deg_kernel(dst_hbm, ones_hbm, zeros_hbm, out_hbm, idx_v, ones_v, acc):
    c = lax.axis_index("c")
    s = lax.axis_index("s")
    wid = c * NS + s
    pltpu.sync_copy(dst_hbm.at[wid], idx_v)
    pltpu.sync_copy(ones_hbm, ones_v)
    pltpu.sync_copy(zeros_hbm, acc.at[pl.ds(s * ZROWS, ZROWS)])
    plsc.subcore_barrier()

    def body(j, carry):
        pltpu.sync_copy(ones_v, acc.at[idx_v.at[j]], add=True)
        return carry

    lax.fori_loop(0, NBLK0, body, 0)
    plsc.subcore_barrier()
    pltpu.sync_copy(acc.at[pl.ds(s * ZROWS, ZROWS)],
                    out_hbm.at[c, pl.ds(s * ZROWS, ZROWS)])


def _deg(dst_pad):
    ones = jnp.ones((128, 128), _f32)
    zeros = jnp.zeros((ZROWS, 128), _f32)
    k = pl.kernel(
        _deg_kernel,
        out_type=jax.ShapeDtypeStruct((NC, NPAD, 128), _f32),
        mesh=_sc_mesh(),
        scratch_types=[
            pltpu.VMEM((NBLK0, 128), _i32),
            pltpu.VMEM((128, 128), _f32),
            pltpu.VMEM_SHARED((NPAD, 128), _f32),
        ],
    )
    return k(dst_pad.reshape(NC * NS, NBLK0, 128), ones, zeros)


def _seg_body(nc, table_hbm, src_hbm, dst_hbm, zeros_hbm, out_hbm,
              srcv, dstv, buf0, buf1, acc, gsem0, gsem1, ssem0, ssem1):
    c = lax.axis_index("c")
    s = lax.axis_index("s")
    pltpu.sync_copy(dst_hbm.at[s], dstv)
    for r in range(nc // NC):
        ch = r * NC + c
        pltpu.sync_copy(zeros_hbm, acc.at[pl.ds(s * ZROWS, ZROWS)])
        plsc.subcore_barrier()
        for half in range(2):
            base = half * HBLK
            pltpu.sync_copy(src_hbm.at[ch, s, half], srcv)
            def gstart(j, buf, semA, semB):
                pltpu.make_async_copy(
                    table_hbm.at[srcv.at[j, pl.ds(0, 64)]],
                    buf.at[pl.ds(0, 64)], semA).start()
                pltpu.make_async_copy(
                    table_hbm.at[srcv.at[j, pl.ds(64, 64)]],
                    buf.at[pl.ds(64, 64)], semB).start()

            def gwait(j, buf, semA, semB):
                pltpu.make_async_copy(
                    table_hbm.at[srcv.at[j, pl.ds(0, 64)]],
                    buf.at[pl.ds(0, 64)], semA).wait()
                pltpu.make_async_copy(
                    table_hbm.at[srcv.at[j, pl.ds(64, 64)]],
                    buf.at[pl.ds(64, 64)], semB).wait()

            gstart(0, buf0, gsem0, ssem0)
            gstart(1, buf1, gsem1, ssem1)

            def body(t, carry):
                j = 2 * t
                gwait(j, buf0, gsem0, ssem0)
                pltpu.sync_copy(buf0, acc.at[dstv.at[base + j]], add=True)

                @pl.when(j + 2 < HBLK)
                def _():
                    gstart(j + 2, buf0, gsem0, ssem0)

                gwait(j + 1, buf1, gsem1, ssem1)
                pltpu.sync_copy(buf1, acc.at[dstv.at[base + j + 1]], add=True)

                @pl.when(j + 3 < HBLK)
                def _():
                    gstart(j + 3, buf1, gsem1, ssem1)

                return carry

            lax.fori_loop(0, HBLK // 2, body, 0)
        plsc.subcore_barrier()
        pltpu.sync_copy(acc.at[pl.ds(s * ZROWS, ZROWS)],
                        out_hbm.at[ch, pl.ds(s * ZROWS, ZROWS)])
        plsc.subcore_barrier()


def _seg(table, src_pad, dst_pad, nc):
    zeros = jnp.zeros((ZROWS, 128), _f32)
    k = pl.kernel(
        functools.partial(_seg_body, nc),
        out_type=jax.ShapeDtypeStruct((nc, NPAD, 128), _f32),
        mesh=_sc_mesh(),
        scratch_types=[
            pltpu.VMEM((HBLK, 128), _i32),
            pltpu.VMEM((NBLK, 128), _i32),
            pltpu.VMEM((128, 128), _f32),
            pltpu.VMEM((128, 128), _f32),
            pltpu.VMEM_SHARED((NPAD, 128), _f32),
            pltpu.SemaphoreType.DMA,
            pltpu.SemaphoreType.DMA,
            pltpu.SemaphoreType.DMA,
            pltpu.SemaphoreType.DMA,
        ],
    )
    return k(table, src_pad.reshape(nc, NS, 2, HBLK, 128),
             dst_pad.reshape(NS, NBLK, 128), zeros)


def _tc1_body(x_ref, hist_ref, wa_ref, ba_ref, t1_ref, dinv_ref):
    hist = hist_ref[...]
    deg = (jnp.sum(hist[0], axis=-1) + jnp.sum(hist[1], axis=-1)) * 0.0078125 + 1.0
    dinv = lax.rsqrt(deg)
    x = x_ref[...]
    a = jnp.dot(x, wa_ref[...], preferred_element_type=_f32,
                precision=lax.Precision.DEFAULT) + ba_ref[...][None, :]
    m = jnp.max(a, axis=-1, keepdims=True)
    e = jnp.exp(a - m)
    att = e / jnp.sum(e, axis=-1, keepdims=True)
    t1_ref[:, 0, :] = x * dinv[:, None]
    t1_ref[:, 1, :] = (x * att) * dinv[:, None]
    dinv_ref[...] = dinv[:, None]


def _tc1(x, hist, w_att, b_att):
    return pl.pallas_call(
        _tc1_body,
        grid=(GRID,),
        in_specs=[
            pl.BlockSpec((BN, D_IN), lambda i: (i, 0)),
            pl.BlockSpec((NC, BN, 128), lambda i: (0, i, 0)),
            pl.BlockSpec((D_IN, D_IN), lambda i: (0, 0)),
            pl.BlockSpec((D_IN,), lambda i: (0,)),
        ],
        out_specs=[
            pl.BlockSpec((BN, 2, D_IN), lambda i: (i, 0, 0)),
            pl.BlockSpec((BN, 1), lambda i: (i, 0)),
        ],
        out_shape=[
            jax.ShapeDtypeStruct((N, 2, D_IN), _f32),
            jax.ShapeDtypeStruct((N, 1), _f32),
        ],
    )(x, hist, w_att, b_att)


def _tc2_body(agg_ref, t1_ref, dinv_ref, w1_ref, b1_ref, t2_ref):
    dinv = dinv_ref[...]
    for i in range(2):
        g = (agg_ref[i] + t1_ref[:, i, :]) * dinv
        h = jnp.dot(g, w1_ref[...], preferred_element_type=_f32,
                    precision=lax.Precision.DEFAULT) + b1_ref[...][None, :]
        h = jnp.maximum(h, 0.0) * dinv
        for cch in range(4):
            t2_ref[:, 4 * i + cch, :] = h[:, cch * 128:(cch + 1) * 128]


def _tc2(agg1, t1, dinv, w1, b1):
    return pl.pallas_call(
        _tc2_body,
        grid=(GRID,),
        in_specs=[
            pl.BlockSpec((2, BN, D_IN), lambda i: (0, i, 0)),
            pl.BlockSpec((BN, 2, D_IN), lambda i: (i, 0, 0)),
            pl.BlockSpec((BN, 1), lambda i: (i, 0)),
            pl.BlockSpec((D_IN, D_H), lambda i: (0, 0)),
            pl.BlockSpec((D_H,), lambda i: (0,)),
        ],
        out_specs=pl.BlockSpec((BN, 8, 128), lambda i: (i, 0, 0)),
        out_shape=jax.ShapeDtypeStruct((N, 8, 128), _f32),
    )(agg1, t1, dinv, w1, b1)


def _tc3_body(agg_ref, t2_ref, dinv_ref, w2_ref, b2_ref, wp1_ref, bp1_ref,
              wp2_ref, bp2_ref, z1_ref, z2_ref):
    dinv = dinv_ref[...]
    for i in range(2):
        cols = [agg_ref[4 * i + cch] + t2_ref[:, 4 * i + cch, :]
                for cch in range(4)]
        g = jnp.concatenate(cols, axis=-1) * dinv
        h2 = jnp.maximum(
            jnp.dot(g, w2_ref[...], preferred_element_type=_f32,
                    precision=lax.Precision.DEFAULT) + b2_ref[...][None, :], 0.0)
        p = jnp.maximum(
            jnp.dot(h2, wp1_ref[...], preferred_element_type=_f32,
                    precision=lax.Precision.DEFAULT) + bp1_ref[...][None, :],
            0.0)
        z = jnp.dot(p, wp2_ref[...], preferred_element_type=_f32,
                    precision=lax.Precision.DEFAULT) + bp2_ref[...][None, :]
        if i == 0:
            z1_ref[...] = z
        else:
            z2_ref[...] = z


def _tc3(agg2, t2, dinv, w2, b2, wp1, bp1, wp2, bp2):
    return pl.pallas_call(
        _tc3_body,
        grid=(GRID,),
        in_specs=[
            pl.BlockSpec((8, BN, 128), lambda i: (0, i, 0)),
            pl.BlockSpec((BN, 8, 128), lambda i: (i, 0, 0)),
            pl.BlockSpec((BN, 1), lambda i: (i, 0)),
            pl.BlockSpec((D_H, D_H), lambda i: (0, 0)),
            pl.BlockSpec((D_H,), lambda i: (0,)),
            pl.BlockSpec((D_H, D_H), lambda i: (0, 0)),
            pl.BlockSpec((D_H,), lambda i: (0,)),
            pl.BlockSpec((D_H, D_H), lambda i: (0, 0)),
            pl.BlockSpec((D_H,), lambda i: (0,)),
        ],
        out_specs=[
            pl.BlockSpec((BN, D_H), lambda i: (i, 0)),
            pl.BlockSpec((BN, D_H), lambda i: (i, 0)),
        ],
        out_shape=[
            jax.ShapeDtypeStruct((N, D_H), _f32),
            jax.ShapeDtypeStruct((N, D_H), _f32),
        ],
    )(agg2, t2, dinv, w2, b2, wp1, bp1, wp2, bp2)


def kernel(x, edge_index, W_att, b_att, W1, b1, W2, b2, Wp1, bp1, Wp2, bp2):
    src = edge_index[0]
    dst = edge_index[1]
    pad = EPAD - E
    dst_pad = jnp.concatenate([dst, jnp.full((pad,), JUNK, _i32)])
    src_pad = jnp.concatenate([src, jnp.zeros((pad,), _i32)])
    idx1 = src_pad[None, :] * 2 + jnp.arange(2, dtype=_i32)[:, None]
    idx2 = src_pad[None, :] * 8 + jnp.arange(8, dtype=_i32)[:, None]

    hist = _deg(dst_pad)
    t1, dinv = _tc1(x, hist, W_att, b_att)
    agg1 = _seg(t1.reshape(N * 2, 128), idx1, dst_pad, 2)
    t2 = _tc2(agg1, t1, dinv, W1, b1)
    agg2 = _seg(t2.reshape(N * 8, 128), idx2, dst_pad, 8)
    z1, z2 = _tc3(agg2, t2, dinv, W2, b2, Wp1, bp1, Wp2, bp2)
    return (z1, z2)
